# baseline (speedup 1.0000x reference)
import functools

import jax
import jax.numpy as jnp
from jax import lax
from jax.experimental import pallas as pl
from jax.experimental.pallas import tpu as pltpu
from jax.experimental.pallas import tpu_sc as plsc

C = 1000
CPAD = 1024
N = 16384
D = 1024
NC, NS = 2, 16
NT = NC * NS
TCOLS = D // NT
K = 128
NCHUNK = N // K
CHUNKS_PER_T = NCHUNK // NT


def _seg_body(emb, lab, out_sums, out_cnt, rowbuf, idxbuf, acc, cnt):
    c = lax.axis_index("c")
    s = lax.axis_index("s")
    t = c * NS + s

    zvec = jnp.zeros((16,), jnp.float32)
    onevec = jnp.ones((16,), jnp.float32)

    def zbody(i, _):
        acc[i, pl.ds(0, 16)] = zvec
        acc[i, pl.ds(16, 16)] = zvec
        cnt[i, :] = zvec
        return 0
    lax.fori_loop(0, CPAD, zbody, 0)

    col0 = t * TCOLS

    def chunk_body(k, _):
        r0 = k * K
        pltpu.sync_copy(lab.at[pl.ds(r0, K)], idxbuf)
        pltpu.sync_copy(emb.at[pl.ds(r0, K), pl.ds(col0, TCOLS)], rowbuf)

        def grp_body(g, _):
            j0 = g * 16
            lblv = idxbuf[pl.ds(j0, 16)]
            for i in range(16):
                lbl = lblv[i]
                j = j0 + i
                plsc.addupdate(acc.at[lbl, pl.ds(0, 16)], rowbuf[j, pl.ds(0, 16)])
                plsc.addupdate(acc.at[lbl, pl.ds(16, 16)], rowbuf[j, pl.ds(16, 16)])
            return 0
        lax.fori_loop(0, K // 16, grp_body, 0)

        @pl.when((k >= t * CHUNKS_PER_T) & (k < (t + 1) * CHUNKS_PER_T))
        def _():
            def cnt_body(g, _):
                lblv = idxbuf[pl.ds(g * 16, 16)]
                for i in range(16):
                    plsc.addupdate(cnt.at[lblv[i]], onevec)
                return 0
            lax.fori_loop(0, K // 16, cnt_body, 0)
        return 0
    lax.fori_loop(0, NCHUNK, chunk_body, 0)

    pltpu.sync_copy(acc, out_sums.at[pl.ds(0, CPAD), pl.ds(col0, TCOLS)])
    pltpu.sync_copy(cnt, out_cnt.at[t])


@functools.lru_cache(maxsize=1)
def _make_seg_kernel():
    mesh = plsc.VectorSubcoreMesh(
        core_axis_name="c", subcore_axis_name="s",
        num_cores=NC, num_subcores=NS)
    return pl.kernel(
        _seg_body,
        out_type=[
            jax.ShapeDtypeStruct((CPAD, D), jnp.float32),
            jax.ShapeDtypeStruct((NT, CPAD, 16), jnp.float32),
        ],
        mesh=mesh,
        compiler_params=pltpu.CompilerParams(use_tc_tiling_on_sc=False),
        scratch_types=[
            pltpu.VMEM((K, TCOLS), jnp.float32),
            pltpu.VMEM((K,), jnp.int32),
            pltpu.VMEM((CPAD, TCOLS), jnp.float32),
            pltpu.VMEM((CPAD, 16), jnp.float32),
        ],
    )


def _finish_body(sums_ref, cnt_ref, ref_ref, out_ref):
    cnt = jnp.sum(cnt_ref[:, :, 0], axis=0)[:, None]
    means = sums_ref[...] / jnp.maximum(cnt, 1.0)
    d = means - ref_ref[...]
    mse = jnp.sum(d * d, axis=1, keepdims=True) * (1.0 / D)
    out_ref[0, 0] = jnp.sum(jnp.where(cnt > 0, mse, 0.0))


def kernel(embeddings, labels, ref_centroids):
    seg = _make_seg_kernel()
    sums, cnt_p = seg(embeddings, labels)
    refpad = jnp.pad(ref_centroids, ((0, CPAD - C), (0, 0)))
    loss = pl.pallas_call(
        _finish_body,
        out_shape=jax.ShapeDtypeStruct((1, 1), jnp.float32),
        out_specs=pl.BlockSpec(memory_space=pltpu.SMEM),
    )(sums, cnt_p, refpad)
    return loss[0, 0]
